# Initial kernel scaffold; baseline (speedup 1.0000x reference)
import jax
import jax.numpy as jnp
from jax.experimental import pallas as pl


def kernel(inputs, which_model, table, params):
    raise NotImplementedError("write your pallas kernel here")


# speedup vs baseline: 1.9169x; 1.9169x over previous
import functools

import jax
import jax.numpy as jnp
from jax import lax
from jax.experimental import pallas as pl
from jax.experimental.pallas import tpu as pltpu
from jax.experimental.pallas import tpu_sc as plsc

ACC = 256
LBAG = 50
NC, NS = 2, 16
NW = NC * NS
BAGS_W = 512
GB = 2
NCH = BAGS_W // GB
IDXC = GB * LBAG
IDXP = 104
FL_CH = 32
FL_ROWS = FL_CH * GB
NV = ACC // 16


def _crelu(x, leak=0.05):
    c = jnp.clip(x, -1.0, 127.0 / 128.0)
    return c + leak * (x - c)


def _sc_embed_sum(table256, idx3):
    mesh = plsc.VectorSubcoreMesh(core_axis_name="c", subcore_axis_name="s",
                                  num_cores=NC, num_subcores=NS)

    @functools.partial(
        pl.kernel,
        out_type=jax.ShapeDtypeStruct((NW * BAGS_W, ACC), jnp.float32),
        mesh=mesh,
        scratch_types=[
            pltpu.VMEM((NCH, IDXP), jnp.int32),
            pltpu.VMEM((IDXP, ACC), jnp.float32),
            pltpu.VMEM((IDXP, ACC), jnp.float32),
            pltpu.VMEM((FL_ROWS, ACC), jnp.float32),
            pltpu.SemaphoreType.DMA,
            pltpu.SemaphoreType.DMA,
        ],
    )
    def sc_kernel(table_hbm, idx_hbm, out_hbm, idx_v, buf0, buf1, stage,
                  sem0, sem1):
        wid = lax.axis_index("s") * NC + lax.axis_index("c")
        pltpu.sync_copy(idx_hbm.at[wid], idx_v)
        bufs = (buf0, buf1)
        sems = (sem0, sem1)

        def start(ch, b):
            pltpu.async_copy(table_hbm.at[idx_v.at[ch]], bufs[b], sems[b])

        def wait(b):
            pltpu.make_async_copy(
                table_hbm.at[idx_v.at[0]], bufs[b], sems[b]).wait()

        def accum(b, slot):
            buf = bufs[b]
            for k in range(GB):
                def body(l, acc, _k=k):
                    return tuple(
                        acc[j] + buf[_k * LBAG + l, pl.ds(16 * j, 16)]
                        for j in range(NV))
                acc = lax.fori_loop(
                    0, LBAG, body,
                    tuple(jnp.zeros((16,), jnp.float32) for _ in range(NV)))
                for j in range(NV):
                    stage[slot + k, pl.ds(16 * j, 16)] = acc[j]

        start(0, 0)
        start(1, 1)

        def step(t, carry):
            tl = lax.rem(t, FL_CH // GB)
            for b in range(GB):
                ch = GB * t + b
                wait(b)
                accum(b, 2 * GB * tl + GB * b)

                @pl.when(ch + GB < NCH)
                def _():
                    start(ch + GB, b)

            @pl.when(tl == FL_CH // GB - 1)
            def _():
                row0 = wid * BAGS_W + (t // (FL_CH // GB)) * FL_ROWS
                pltpu.sync_copy(stage, out_hbm.at[pl.ds(row0, FL_ROWS)])
            return carry

        lax.fori_loop(0, NCH // GB, step, 0)

    return sc_kernel(table256, idx3)


def _tc_mlp(sums, which2d, w1, b1, w2, b2, w3, b3):
    R = 512
    Bb = sums.shape[0] // R

    def body(s_ref, wm_ref, w1_ref, b1_ref, w2_ref, b2_ref, w3_ref, b3_ref,
             o_ref):
        x = _crelu(s_ref[...] * (1.0 / LBAG))
        wm = wm_ref[0, 0, :]
        cols = []
        for n in range(4):
            h1 = _crelu(
                lax.dot_general(x, w1_ref[n], (((1,), (1,)), ((), ())),
                                preferred_element_type=jnp.float32)
                + b1_ref[n])
            h2 = _crelu(
                lax.dot_general(h1, w2_ref[n], (((1,), (1,)), ((), ())),
                                preferred_element_type=jnp.float32)
                + b2_ref[n])
            cols.append(h2)
        hcat = jnp.concatenate(cols, axis=1)
        outs = lax.dot_general(hcat, w3_ref[...],
                               (((1,), (0,)), ((), ())),
                               preferred_element_type=jnp.float32)
        outs = outs + b3_ref[...]
        onehot = (wm[:, None]
                  == lax.broadcasted_iota(jnp.int32, (1, 4), 1)
                  ).astype(jnp.float32)
        val = jnp.sum(outs * onehot, axis=1)
        o_ref[0, 0, :] = jnp.tanh(val)

    zero = lambda i: (0, 0)
    zero3 = lambda i: (0, 0, 0)
    return pl.pallas_call(
        body,
        grid=(Bb,),
        in_specs=[
            pl.BlockSpec((R, ACC), lambda i: (i, 0)),
            pl.BlockSpec((1, 1, R), lambda i: (i, 0, 0)),
            pl.BlockSpec((4, 16, ACC), zero3),
            pl.BlockSpec((4, 16), zero),
            pl.BlockSpec((4, 32, 16), zero3),
            pl.BlockSpec((4, 32), zero),
            pl.BlockSpec((128, 4), zero),
            pl.BlockSpec((1, 4), zero),
        ],
        out_specs=pl.BlockSpec((1, 1, R), lambda i: (i, 0, 0)),
        out_shape=jax.ShapeDtypeStruct((Bb, 1, R), jnp.float32),
    )(sums, which2d, w1, b1, w2, b2, w3, b3)


_HEADS = ['white_main', 'black_main', 'white_duck', 'black_duck']


def kernel(inputs, which_model, table, params):
    B = inputs.shape[0]
    table256 = table[:, :ACC]
    idx3 = jnp.pad(inputs.reshape(NW, NCH, IDXC),
                   ((0, 0), (0, 0), (0, IDXP - IDXC)))
    sums = _sc_embed_sum(table256, idx3)

    w1 = jnp.stack([params[n]['W1'] for n in _HEADS])
    b1 = jnp.stack([params[n]['b1'] for n in _HEADS])
    w2 = jnp.stack([params[n]['W2'] for n in _HEADS])
    b2 = jnp.stack([params[n]['b2'] for n in _HEADS])
    w3cat = jnp.stack([params[n]['W3'][0] for n in _HEADS])
    w3 = (w3cat[:, :, None]
          * jnp.eye(4, dtype=jnp.float32)[:, None, :]).reshape(128, 4)
    b3 = jnp.stack([params[n]['b3'][0] for n in _HEADS])[None]

    R = 512
    which2d = which_model.reshape(B // R, 1, R)
    vals = _tc_mlp(sums, which2d, w1, b1, w2, b2, w3, b3)
    return vals.reshape(B, 1)
